# baseline (speedup 1.0000x reference)
import jax
import jax.numpy as jnp
from jax import lax
from jax.experimental import pallas as pl
from jax.experimental.pallas import tpu as pltpu
from jax.experimental.pallas import tpu_sc as plsc

_NC = 2
_NS = 16
_NW = _NC * _NS

_HALF = 5120
_FULL = 2 * _HALF
_LSTRIDE = _HALF + 1

_RB = 1000


def _sc_mesh():
    return plsc.VectorSubcoreMesh(core_axis_name="c", subcore_axis_name="s",
                                  num_cores=_NC, num_subcores=_NS)


def _deg_body(dst_hbm, zeros_hbm, out_hbm, dst_v, hist_v, deg_v):
    cid = lax.axis_index("c")
    sid = lax.axis_index("s")
    wid = sid * _NC + cid
    epw = dst_v.shape[0]

    pltpu.sync_copy(dst_hbm.at[wid], dst_v)

    lane_base = lax.iota(jnp.int32, 16) * _LSTRIDE
    ones = jnp.ones((16,), jnp.float32)

    for p in range(2):
        lo = p * _HALF
        pltpu.sync_copy(zeros_hbm, hist_v)

        def body(i, carry, lo=lo):
            dvec = dst_v[pl.ds(i * 16, 16)]
            m = (dvec >= lo) & (dvec < lo + _HALF)
            idx = jnp.where(m, dvec - lo, 0) + lane_base
            plsc.addupdate_scatter(hist_v, [idx], ones, mask=m)
            return carry

        lax.fori_loop(0, epw // 16, body, 0)

        def red(q, carry):
            s = hist_v[pl.ds(q * 16, 16)]
            for l in range(1, 16):
                s = s + hist_v[pl.ds(l * _LSTRIDE + q * 16, 16)]
            deg_v[pl.ds(q * 16, 16)] = s
            return carry

        lax.fori_loop(0, _HALF // 16, red, 0)
        pltpu.sync_copy(deg_v, out_hbm.at[wid, pl.ds(p * _HALF, _HALF)])


_NBUF = 8


def _acc_body(g_hbm, src_hbm, dst_hbm, zeros_hbm, out_hbm,
              src_v, dst_v, bufs, gsems, ssems, shared_acc):
    npad = shared_acc.shape[0]
    nchunk = src_v.shape[0]
    cid = lax.axis_index("c")
    sid = lax.axis_index("s")
    wid = sid * _NC + cid
    rpt = npad // _NS
    r0 = pl.multiple_of(sid * rpt, 8)

    pltpu.sync_copy(src_hbm.at[wid], src_v)
    pltpu.sync_copy(dst_hbm.at[wid], dst_v)
    pltpu.sync_copy(zeros_hbm.at[pl.ds(r0, rpt)], shared_acc.at[pl.ds(r0, rpt)])
    plsc.subcore_barrier()

    def gather(c, b):
        pltpu.async_copy(g_hbm.at[src_v.at[c]], bufs[b], gsems[b])

    def wait_gather(c, b):
        pltpu.make_async_copy(g_hbm.at[src_v.at[c]], bufs[b], gsems[b]).wait()

    def scatter(c, b):
        pltpu.async_copy(bufs[b], shared_acc.at[dst_v.at[c]], ssems[b],
                         add=True)

    def wait_scatter(c, b):
        pltpu.make_async_copy(bufs[b], shared_acc.at[dst_v.at[c]],
                              ssems[b]).wait()

    depth = _NBUF // 2
    for c in range(depth):
        gather(c, c)

    def outer(gidx, carry):
        for b in range(_NBUF):
            c = gidx * _NBUF + b
            wait_gather(c, b)
            scatter(c, b)

            @pl.when(c >= depth)
            def _(c=c, b=b):
                wait_scatter(c - depth, (b + depth) % _NBUF)

            @pl.when(c + depth < nchunk)
            def _(c=c, b=b):
                gather(c + depth, (b + depth) % _NBUF)
        return carry

    lax.fori_loop(0, nchunk // _NBUF, outer, 0)
    for c in range(nchunk - depth, nchunk):
        wait_scatter(c, c % _NBUF)

    plsc.subcore_barrier()
    pltpu.sync_copy(shared_acc.at[pl.ds(r0, rpt)],
                    out_hbm.at[cid, pl.ds(r0, rpt)])


def _make_acc_call(dc, nchunk, ch):
    return pl.kernel(
        _acc_body,
        out_type=jax.ShapeDtypeStruct((_NC, _FULL, dc), jnp.bfloat16),
        mesh=_sc_mesh(),
        scratch_types=[
            pltpu.VMEM((nchunk, ch), jnp.int32),
            pltpu.VMEM((nchunk, ch), jnp.int32),
            tuple(pltpu.VMEM((ch, dc), jnp.bfloat16) for _ in range(_NBUF)),
            tuple(pltpu.SemaphoreType.DMA for _ in range(_NBUF)),
            tuple(pltpu.SemaphoreType.DMA for _ in range(_NBUF)),
            pltpu.VMEM_SHARED((_FULL, dc), jnp.bfloat16),
        ],
        compiler_params=pltpu.CompilerParams(needs_layout_passes=False,
                                             use_tc_tiling_on_sc=False),
    )


def _tc_h1_body(x_ref, w_ref, h_ref):
    h_ref[...] = jnp.dot(x_ref[...], w_ref[...],
                         preferred_element_type=jnp.float32)


def _tc_first_body(deg_ref, h_ref, g_ref, inv_ref):
    deg = jnp.sum(deg_ref[...], axis=1) + 1.0
    inv = lax.rsqrt(deg)[:, None]
    inv_ref[...] = inv
    g_ref[...] = (h_ref[...] * inv).astype(jnp.bfloat16)


def _tc_mid_body(acc_ref, g_ref, inv_ref, b_ref, w_ref, o_ref):
    inv = inv_ref[...]
    s = (acc_ref[0].astype(jnp.float32) + acc_ref[1].astype(jnp.float32)
         + g_ref[...].astype(jnp.float32))
    x = jnp.maximum(inv * s + b_ref[...], 0.0)
    h = jnp.dot(x * inv, w_ref[...], preferred_element_type=jnp.float32)
    o_ref[...] = h.astype(jnp.bfloat16)


def _tc_final_body(acc_ref, g_ref, inv_ref, b_ref, o_ref):
    s = (acc_ref[0].astype(jnp.float32) + acc_ref[1].astype(jnp.float32)
         + g_ref[...].astype(jnp.float32))
    o_ref[...] = inv_ref[...] * s + b_ref[...]


def _tc_h1(x, W):
    n, d_in = x.shape
    d_out = W.shape[1]
    return pl.pallas_call(
        _tc_h1_body,
        grid=(n // _RB,),
        in_specs=[
            pl.BlockSpec((_RB, d_in), lambda i: (i, 0)),
            pl.BlockSpec((d_in, d_out), lambda i: (0, 0)),
        ],
        out_specs=pl.BlockSpec((_RB, d_out), lambda i: (i, 0)),
        out_shape=jax.ShapeDtypeStruct((n, d_out), jnp.float32),
    )(x, W)


def _tc_first(deg_p, h1):
    n, d_out = h1.shape
    return pl.pallas_call(
        _tc_first_body,
        grid=(n // _RB,),
        in_specs=[
            pl.BlockSpec((_RB, _NW), lambda i: (i, 0)),
            pl.BlockSpec((_RB, d_out), lambda i: (i, 0)),
        ],
        out_specs=[
            pl.BlockSpec((_RB, d_out), lambda i: (i, 0)),
            pl.BlockSpec((_RB, 1), lambda i: (i, 0)),
        ],
        out_shape=[
            jax.ShapeDtypeStruct((n, d_out), jnp.bfloat16),
            jax.ShapeDtypeStruct((n, 1), jnp.float32),
        ],
    )(deg_p, h1)


def _tc_mid(acc_p, g, inv, b, W):
    n, d = g.shape
    d_out = W.shape[1]
    return pl.pallas_call(
        _tc_mid_body,
        grid=(n // _RB,),
        in_specs=[
            pl.BlockSpec((_NC, _RB, d), lambda i: (0, i, 0)),
            pl.BlockSpec((_RB, d), lambda i: (i, 0)),
            pl.BlockSpec((_RB, 1), lambda i: (i, 0)),
            pl.BlockSpec((1, d), lambda i: (0, 0)),
            pl.BlockSpec((d, d_out), lambda i: (0, 0)),
        ],
        out_specs=pl.BlockSpec((_RB, d_out), lambda i: (i, 0)),
        out_shape=jax.ShapeDtypeStruct((n, d_out), jnp.bfloat16),
    )(acc_p, g, inv, b, W)


def _tc_final(acc_p, g, inv, b):
    n, d = g.shape
    return pl.pallas_call(
        _tc_final_body,
        grid=(n // _RB,),
        in_specs=[
            pl.BlockSpec((_NC, _RB, d), lambda i: (0, i, 0)),
            pl.BlockSpec((_RB, d), lambda i: (i, 0)),
            pl.BlockSpec((_RB, 1), lambda i: (i, 0)),
            pl.BlockSpec((1, d), lambda i: (0, 0)),
        ],
        out_specs=pl.BlockSpec((_RB, d), lambda i: (i, 0)),
        out_shape=jax.ShapeDtypeStruct((n, d), jnp.float32),
    )(acc_p, g, inv, b)


def kernel(node_features, edge_index, W1, b1, W2, b2, W3, b3):
    n, d_in = node_features.shape
    e = edge_index.shape[1]
    epw = e // _NW
    ch = 125
    nchunk = epw // ch

    src = edge_index[0].reshape(_NW, nchunk, ch)
    dst = edge_index[1].reshape(_NW, nchunk, ch)
    dst_flat = edge_index[1].reshape(_NW, epw)
    zeros_deg = jnp.zeros((_LSTRIDE * 16,), jnp.float32)
    zeros_h = jnp.zeros((_FULL, W1.shape[1]), jnp.bfloat16)
    zeros_o = jnp.zeros((_FULL, W3.shape[1]), jnp.bfloat16)

    deg_call = pl.kernel(
        _deg_body,
        out_type=jax.ShapeDtypeStruct((_NW, _FULL), jnp.float32),
        mesh=_sc_mesh(),
        scratch_types=[
            pltpu.VMEM((epw,), jnp.int32),
            pltpu.VMEM((_LSTRIDE * 16,), jnp.float32),
            pltpu.VMEM((_HALF,), jnp.float32),
        ],
        compiler_params=pltpu.CompilerParams(needs_layout_passes=False),
    )
    deg_p = deg_call(dst_flat, zeros_deg).T

    acc_h = _make_acc_call(W1.shape[1], nchunk, ch)
    acc_o = _make_acc_call(W3.shape[1], nchunk, ch)

    h1 = _tc_h1(node_features, W1)
    g1, inv = _tc_first(deg_p, h1)
    a1 = acc_h(g1, src, dst, zeros_h)
    g2 = _tc_mid(a1, g1, inv, b1.reshape(1, -1), W2)
    a2 = acc_h(g2, src, dst, zeros_h)
    g3 = _tc_mid(a2, g2, inv, b2.reshape(1, -1), W3)
    a3 = acc_o(g3, src, dst, zeros_o)
    return _tc_final(a3, g3, inv, b3.reshape(1, -1))


# speedup vs baseline: 1.0017x; 1.0017x over previous
import jax
import jax.numpy as jnp
from jax import lax
from jax.experimental import pallas as pl
from jax.experimental.pallas import tpu as pltpu
from jax.experimental.pallas import tpu_sc as plsc

_NC = 2
_NS = 16
_NW = _NC * _NS

_HALF = 5120
_FULL = 2 * _HALF
_LSTRIDE = _HALF + 1

_RB = 1000


def _sc_mesh():
    return plsc.VectorSubcoreMesh(core_axis_name="c", subcore_axis_name="s",
                                  num_cores=_NC, num_subcores=_NS)


def _deg_body(dst_hbm, zeros_hbm, out_hbm, dst_v, hist_v, deg_v):
    cid = lax.axis_index("c")
    sid = lax.axis_index("s")
    wid = sid * _NC + cid
    epw = dst_v.shape[0]

    pltpu.sync_copy(dst_hbm.at[wid], dst_v)

    lane_base = lax.iota(jnp.int32, 16) * _LSTRIDE
    ones = jnp.ones((16,), jnp.float32)

    for p in range(2):
        lo = p * _HALF
        pltpu.sync_copy(zeros_hbm, hist_v)

        def hist16(base, lo=lo):
            dvec = dst_v[pl.ds(base, 16)]
            m = (dvec >= lo) & (dvec < lo + _HALF)
            idx = jnp.where(m, dvec - lo, 0) + lane_base
            plsc.addupdate_scatter(hist_v, [idx], ones, mask=m)

        def body(i, carry):
            for u in range(4):
                hist16(i * 64 + u * 16)
            return carry

        lax.fori_loop(0, epw // 64, body, 0)
        for t in range((epw // 64) * 64, epw, 16):
            hist16(t)

        def red(q, carry):
            s = hist_v[pl.ds(q * 16, 16)]
            for l in range(1, 16):
                s = s + hist_v[pl.ds(l * _LSTRIDE + q * 16, 16)]
            deg_v[pl.ds(q * 16, 16)] = s
            return carry

        lax.fori_loop(0, _HALF // 16, red, 0)
        pltpu.sync_copy(deg_v, out_hbm.at[wid, pl.ds(p * _HALF, _HALF)])


_NBUF = 8


def _acc_body(g_hbm, src_hbm, dst_hbm, zeros_hbm, out_hbm,
              src_v, dst_v, bufs, gsems, ssems, shared_acc):
    npad = shared_acc.shape[0]
    nchunk = src_v.shape[0]
    cid = lax.axis_index("c")
    sid = lax.axis_index("s")
    wid = sid * _NC + cid
    rpt = npad // _NS
    r0 = pl.multiple_of(sid * rpt, 8)

    pltpu.sync_copy(src_hbm.at[wid], src_v)
    pltpu.sync_copy(dst_hbm.at[wid], dst_v)
    pltpu.sync_copy(zeros_hbm.at[pl.ds(r0, rpt)], shared_acc.at[pl.ds(r0, rpt)])
    plsc.subcore_barrier()

    def gather(c, b):
        pltpu.async_copy(g_hbm.at[src_v.at[c]], bufs[b], gsems[b])

    def wait_gather(c, b):
        pltpu.make_async_copy(g_hbm.at[src_v.at[c]], bufs[b], gsems[b]).wait()

    def scatter(c, b):
        pltpu.async_copy(bufs[b], shared_acc.at[dst_v.at[c]], ssems[b],
                         add=True)

    def wait_scatter(c, b):
        pltpu.make_async_copy(bufs[b], shared_acc.at[dst_v.at[c]],
                              ssems[b]).wait()

    depth = _NBUF // 2
    for c in range(depth):
        gather(c, c)

    def outer(gidx, carry):
        for b in range(_NBUF):
            c = gidx * _NBUF + b
            wait_gather(c, b)
            scatter(c, b)

            @pl.when(c >= depth)
            def _(c=c, b=b):
                wait_scatter(c - depth, (b + depth) % _NBUF)

            @pl.when(c + depth < nchunk)
            def _(c=c, b=b):
                gather(c + depth, (b + depth) % _NBUF)
        return carry

    lax.fori_loop(0, nchunk // _NBUF, outer, 0)
    for c in range(nchunk - depth, nchunk):
        wait_scatter(c, c % _NBUF)

    plsc.subcore_barrier()
    pltpu.sync_copy(shared_acc.at[pl.ds(r0, rpt)],
                    out_hbm.at[cid, pl.ds(r0, rpt)])


def _make_acc_call(dc, nchunk, ch):
    return pl.kernel(
        _acc_body,
        out_type=jax.ShapeDtypeStruct((_NC, _FULL, dc), jnp.bfloat16),
        mesh=_sc_mesh(),
        scratch_types=[
            pltpu.VMEM((nchunk, ch), jnp.int32),
            pltpu.VMEM((nchunk, ch), jnp.int32),
            tuple(pltpu.VMEM((ch, dc), jnp.bfloat16) for _ in range(_NBUF)),
            tuple(pltpu.SemaphoreType.DMA for _ in range(_NBUF)),
            tuple(pltpu.SemaphoreType.DMA for _ in range(_NBUF)),
            pltpu.VMEM_SHARED((_FULL, dc), jnp.bfloat16),
        ],
        compiler_params=pltpu.CompilerParams(needs_layout_passes=False,
                                             use_tc_tiling_on_sc=False),
    )


def _tc_h1_body(x_ref, w_ref, h_ref):
    h_ref[...] = jnp.dot(x_ref[...], w_ref[...],
                         preferred_element_type=jnp.float32)


def _tc_first_body(deg_ref, h_ref, g_ref, inv_ref):
    deg = jnp.sum(deg_ref[...], axis=1) + 1.0
    inv = lax.rsqrt(deg)[:, None]
    inv_ref[...] = inv
    g_ref[...] = (h_ref[...] * inv).astype(jnp.bfloat16)


def _tc_mid_body(acc_ref, g_ref, inv_ref, b_ref, w_ref, o_ref):
    inv = inv_ref[...]
    s = (acc_ref[0].astype(jnp.float32) + acc_ref[1].astype(jnp.float32)
         + g_ref[...].astype(jnp.float32))
    x = jnp.maximum(inv * s + b_ref[...], 0.0)
    h = jnp.dot(x * inv, w_ref[...], preferred_element_type=jnp.float32)
    o_ref[...] = h.astype(jnp.bfloat16)


def _tc_final_body(acc_ref, g_ref, inv_ref, b_ref, o_ref):
    s = (acc_ref[0].astype(jnp.float32) + acc_ref[1].astype(jnp.float32)
         + g_ref[...].astype(jnp.float32))
    o_ref[...] = inv_ref[...] * s + b_ref[...]


def _tc_h1(x, W):
    n, d_in = x.shape
    d_out = W.shape[1]
    return pl.pallas_call(
        _tc_h1_body,
        grid=(n // _RB,),
        in_specs=[
            pl.BlockSpec((_RB, d_in), lambda i: (i, 0)),
            pl.BlockSpec((d_in, d_out), lambda i: (0, 0)),
        ],
        out_specs=pl.BlockSpec((_RB, d_out), lambda i: (i, 0)),
        out_shape=jax.ShapeDtypeStruct((n, d_out), jnp.float32),
    )(x, W)


def _tc_first(deg_p, h1):
    n, d_out = h1.shape
    return pl.pallas_call(
        _tc_first_body,
        grid=(n // _RB,),
        in_specs=[
            pl.BlockSpec((_RB, _NW), lambda i: (i, 0)),
            pl.BlockSpec((_RB, d_out), lambda i: (i, 0)),
        ],
        out_specs=[
            pl.BlockSpec((_RB, d_out), lambda i: (i, 0)),
            pl.BlockSpec((_RB, 1), lambda i: (i, 0)),
        ],
        out_shape=[
            jax.ShapeDtypeStruct((n, d_out), jnp.bfloat16),
            jax.ShapeDtypeStruct((n, 1), jnp.float32),
        ],
    )(deg_p, h1)


def _tc_mid(acc_p, g, inv, b, W):
    n, d = g.shape
    d_out = W.shape[1]
    return pl.pallas_call(
        _tc_mid_body,
        grid=(n // _RB,),
        in_specs=[
            pl.BlockSpec((_NC, _RB, d), lambda i: (0, i, 0)),
            pl.BlockSpec((_RB, d), lambda i: (i, 0)),
            pl.BlockSpec((_RB, 1), lambda i: (i, 0)),
            pl.BlockSpec((1, d), lambda i: (0, 0)),
            pl.BlockSpec((d, d_out), lambda i: (0, 0)),
        ],
        out_specs=pl.BlockSpec((_RB, d_out), lambda i: (i, 0)),
        out_shape=jax.ShapeDtypeStruct((n, d_out), jnp.bfloat16),
    )(acc_p, g, inv, b, W)


def _tc_final(acc_p, g, inv, b):
    n, d = g.shape
    return pl.pallas_call(
        _tc_final_body,
        grid=(n // _RB,),
        in_specs=[
            pl.BlockSpec((_NC, _RB, d), lambda i: (0, i, 0)),
            pl.BlockSpec((_RB, d), lambda i: (i, 0)),
            pl.BlockSpec((_RB, 1), lambda i: (i, 0)),
            pl.BlockSpec((1, d), lambda i: (0, 0)),
        ],
        out_specs=pl.BlockSpec((_RB, d), lambda i: (i, 0)),
        out_shape=jax.ShapeDtypeStruct((n, d), jnp.float32),
    )(acc_p, g, inv, b)


def kernel(node_features, edge_index, W1, b1, W2, b2, W3, b3):
    n, d_in = node_features.shape
    e = edge_index.shape[1]
    epw = e // _NW
    ch = 125
    nchunk = epw // ch

    src = edge_index[0].reshape(_NW, nchunk, ch)
    dst = edge_index[1].reshape(_NW, nchunk, ch)
    dst_flat = edge_index[1].reshape(_NW, epw)
    zeros_deg = jnp.zeros((_LSTRIDE * 16,), jnp.float32)
    zeros_h = jnp.zeros((_FULL, W1.shape[1]), jnp.bfloat16)
    zeros_o = jnp.zeros((_FULL, W3.shape[1]), jnp.bfloat16)

    deg_call = pl.kernel(
        _deg_body,
        out_type=jax.ShapeDtypeStruct((_NW, _FULL), jnp.float32),
        mesh=_sc_mesh(),
        scratch_types=[
            pltpu.VMEM((epw,), jnp.int32),
            pltpu.VMEM((_LSTRIDE * 16,), jnp.float32),
            pltpu.VMEM((_HALF,), jnp.float32),
        ],
        compiler_params=pltpu.CompilerParams(needs_layout_passes=False),
    )
    deg_p = deg_call(dst_flat, zeros_deg).T

    acc_h = _make_acc_call(W1.shape[1], nchunk, ch)
    acc_o = _make_acc_call(W3.shape[1], nchunk, ch)

    h1 = _tc_h1(node_features, W1)
    g1, inv = _tc_first(deg_p, h1)
    a1 = acc_h(g1, src, dst, zeros_h)
    g2 = _tc_mid(a1, g1, inv, b1.reshape(1, -1), W2)
    a2 = acc_h(g2, src, dst, zeros_h)
    g3 = _tc_mid(a2, g2, inv, b2.reshape(1, -1), W3)
    a3 = acc_o(g3, src, dst, zeros_o)
    return _tc_final(a3, g3, inv, b3.reshape(1, -1))


# speedup vs baseline: 1.0034x; 1.0017x over previous
import jax
import jax.numpy as jnp
from jax import lax
from jax.experimental import pallas as pl
from jax.experimental.pallas import tpu as pltpu
from jax.experimental.pallas import tpu_sc as plsc

_NC = 2
_NS = 16
_NW = _NC * _NS

_HALF = 5120
_FULL = 2 * _HALF
_LSTRIDE = _HALF + 1

_RB = 1000


def _sc_mesh():
    return plsc.VectorSubcoreMesh(core_axis_name="c", subcore_axis_name="s",
                                  num_cores=_NC, num_subcores=_NS)


def _deg_body(dst_hbm, zeros_hbm, out_hbm, dst_v, hist_v, deg_v):
    cid = lax.axis_index("c")
    sid = lax.axis_index("s")
    wid = sid * _NC + cid
    epw = dst_v.shape[0]

    pltpu.sync_copy(dst_hbm.at[wid], dst_v)

    lane_base = lax.iota(jnp.int32, 16) * _LSTRIDE
    ones = jnp.ones((16,), jnp.float32)

    for p in range(2):
        lo = p * _HALF
        pltpu.sync_copy(zeros_hbm, hist_v)

        def hist16(base, lo=lo):
            dvec = dst_v[pl.ds(base, 16)]
            m = (dvec >= lo) & (dvec < lo + _HALF)
            idx = jnp.where(m, dvec - lo, 0) + lane_base
            plsc.addupdate_scatter(hist_v, [idx], ones, mask=m)

        def body(i, carry):
            for u in range(4):
                hist16(i * 64 + u * 16)
            return carry

        lax.fori_loop(0, epw // 64, body, 0)
        for t in range((epw // 64) * 64, epw, 16):
            hist16(t)

        def red(q, carry):
            vals = [hist_v[pl.ds(l * _LSTRIDE + q * 16, 16)]
                    for l in range(16)]
            while len(vals) > 1:
                vals = [vals[i] + vals[i + 1] for i in range(0, len(vals), 2)]
            deg_v[pl.ds(q * 16, 16)] = vals[0]
            return carry

        lax.fori_loop(0, _HALF // 16, red, 0)
        pltpu.sync_copy(deg_v, out_hbm.at[wid, pl.ds(p * _HALF, _HALF)])


_NBUF = 8


def _acc_body(g_hbm, src_hbm, dst_hbm, zeros_hbm, out_hbm,
              src_v, dst_v, bufs, gsems, ssems, shared_acc):
    npad = shared_acc.shape[0]
    nchunk = src_v.shape[0]
    cid = lax.axis_index("c")
    sid = lax.axis_index("s")
    wid = sid * _NC + cid
    rpt = npad // _NS
    r0 = pl.multiple_of(sid * rpt, 8)

    pltpu.sync_copy(src_hbm.at[wid], src_v)
    pltpu.sync_copy(dst_hbm.at[wid], dst_v)
    pltpu.sync_copy(zeros_hbm.at[pl.ds(r0, rpt)], shared_acc.at[pl.ds(r0, rpt)])
    plsc.subcore_barrier()

    def gather(c, b):
        pltpu.async_copy(g_hbm.at[src_v.at[c]], bufs[b], gsems[b])

    def wait_gather(c, b):
        pltpu.make_async_copy(g_hbm.at[src_v.at[c]], bufs[b], gsems[b]).wait()

    def scatter(c, b):
        pltpu.async_copy(bufs[b], shared_acc.at[dst_v.at[c]], ssems[b],
                         add=True)

    def wait_scatter(c, b):
        pltpu.make_async_copy(bufs[b], shared_acc.at[dst_v.at[c]],
                              ssems[b]).wait()

    depth = _NBUF // 2
    for c in range(depth):
        gather(c, c)

    def outer(gidx, carry):
        for b in range(_NBUF):
            c = gidx * _NBUF + b
            wait_gather(c, b)
            scatter(c, b)

            @pl.when(c >= depth)
            def _(c=c, b=b):
                wait_scatter(c - depth, (b + depth) % _NBUF)

            @pl.when(c + depth < nchunk)
            def _(c=c, b=b):
                gather(c + depth, (b + depth) % _NBUF)
        return carry

    lax.fori_loop(0, nchunk // _NBUF, outer, 0)
    for c in range(nchunk - depth, nchunk):
        wait_scatter(c, c % _NBUF)

    plsc.subcore_barrier()
    pltpu.sync_copy(shared_acc.at[pl.ds(r0, rpt)],
                    out_hbm.at[cid, pl.ds(r0, rpt)])


def _make_acc_call(dc, nchunk, ch):
    return pl.kernel(
        _acc_body,
        out_type=jax.ShapeDtypeStruct((_NC, _FULL, dc), jnp.bfloat16),
        mesh=_sc_mesh(),
        scratch_types=[
            pltpu.VMEM((nchunk, ch), jnp.int32),
            pltpu.VMEM((nchunk, ch), jnp.int32),
            tuple(pltpu.VMEM((ch, dc), jnp.bfloat16) for _ in range(_NBUF)),
            tuple(pltpu.SemaphoreType.DMA for _ in range(_NBUF)),
            tuple(pltpu.SemaphoreType.DMA for _ in range(_NBUF)),
            pltpu.VMEM_SHARED((_FULL, dc), jnp.bfloat16),
        ],
        compiler_params=pltpu.CompilerParams(needs_layout_passes=False,
                                             use_tc_tiling_on_sc=False),
    )


def _tc_h1_body(x_ref, w_ref, h_ref):
    h_ref[...] = jnp.dot(x_ref[...], w_ref[...],
                         preferred_element_type=jnp.float32)


def _tc_first_body(deg_ref, h_ref, g_ref, inv_ref):
    deg = jnp.sum(deg_ref[...], axis=1) + 1.0
    inv = lax.rsqrt(deg)[:, None]
    inv_ref[...] = inv
    g_ref[...] = (h_ref[...] * inv).astype(jnp.bfloat16)


def _tc_mid_body(acc_ref, g_ref, inv_ref, b_ref, w_ref, o_ref):
    inv = inv_ref[...]
    s = (acc_ref[0].astype(jnp.float32) + acc_ref[1].astype(jnp.float32)
         + g_ref[...].astype(jnp.float32))
    x = jnp.maximum(inv * s + b_ref[...], 0.0)
    h = jnp.dot(x * inv, w_ref[...], preferred_element_type=jnp.float32)
    o_ref[...] = h.astype(jnp.bfloat16)


def _tc_final_body(acc_ref, g_ref, inv_ref, b_ref, o_ref):
    s = (acc_ref[0].astype(jnp.float32) + acc_ref[1].astype(jnp.float32)
         + g_ref[...].astype(jnp.float32))
    o_ref[...] = inv_ref[...] * s + b_ref[...]


def _tc_h1(x, W):
    n, d_in = x.shape
    d_out = W.shape[1]
    return pl.pallas_call(
        _tc_h1_body,
        grid=(n // _RB,),
        in_specs=[
            pl.BlockSpec((_RB, d_in), lambda i: (i, 0)),
            pl.BlockSpec((d_in, d_out), lambda i: (0, 0)),
        ],
        out_specs=pl.BlockSpec((_RB, d_out), lambda i: (i, 0)),
        out_shape=jax.ShapeDtypeStruct((n, d_out), jnp.float32),
    )(x, W)


def _tc_first(deg_p, h1):
    n, d_out = h1.shape
    return pl.pallas_call(
        _tc_first_body,
        grid=(n // _RB,),
        in_specs=[
            pl.BlockSpec((_RB, _NW), lambda i: (i, 0)),
            pl.BlockSpec((_RB, d_out), lambda i: (i, 0)),
        ],
        out_specs=[
            pl.BlockSpec((_RB, d_out), lambda i: (i, 0)),
            pl.BlockSpec((_RB, 1), lambda i: (i, 0)),
        ],
        out_shape=[
            jax.ShapeDtypeStruct((n, d_out), jnp.bfloat16),
            jax.ShapeDtypeStruct((n, 1), jnp.float32),
        ],
    )(deg_p, h1)


def _tc_mid(acc_p, g, inv, b, W):
    n, d = g.shape
    d_out = W.shape[1]
    return pl.pallas_call(
        _tc_mid_body,
        grid=(n // _RB,),
        in_specs=[
            pl.BlockSpec((_NC, _RB, d), lambda i: (0, i, 0)),
            pl.BlockSpec((_RB, d), lambda i: (i, 0)),
            pl.BlockSpec((_RB, 1), lambda i: (i, 0)),
            pl.BlockSpec((1, d), lambda i: (0, 0)),
            pl.BlockSpec((d, d_out), lambda i: (0, 0)),
        ],
        out_specs=pl.BlockSpec((_RB, d_out), lambda i: (i, 0)),
        out_shape=jax.ShapeDtypeStruct((n, d_out), jnp.bfloat16),
    )(acc_p, g, inv, b, W)


def _tc_final(acc_p, g, inv, b):
    n, d = g.shape
    return pl.pallas_call(
        _tc_final_body,
        grid=(n // _RB,),
        in_specs=[
            pl.BlockSpec((_NC, _RB, d), lambda i: (0, i, 0)),
            pl.BlockSpec((_RB, d), lambda i: (i, 0)),
            pl.BlockSpec((_RB, 1), lambda i: (i, 0)),
            pl.BlockSpec((1, d), lambda i: (0, 0)),
        ],
        out_specs=pl.BlockSpec((_RB, d), lambda i: (i, 0)),
        out_shape=jax.ShapeDtypeStruct((n, d), jnp.float32),
    )(acc_p, g, inv, b)


def kernel(node_features, edge_index, W1, b1, W2, b2, W3, b3):
    n, d_in = node_features.shape
    e = edge_index.shape[1]
    epw = e // _NW
    ch = 125
    nchunk = epw // ch

    src = edge_index[0].reshape(_NW, nchunk, ch)
    dst = edge_index[1].reshape(_NW, nchunk, ch)
    dst_flat = edge_index[1].reshape(_NW, epw)
    zeros_deg = jnp.zeros((_LSTRIDE * 16,), jnp.float32)
    zeros_h = jnp.zeros((_FULL, W1.shape[1]), jnp.bfloat16)
    zeros_o = jnp.zeros((_FULL, W3.shape[1]), jnp.bfloat16)

    deg_call = pl.kernel(
        _deg_body,
        out_type=jax.ShapeDtypeStruct((_NW, _FULL), jnp.float32),
        mesh=_sc_mesh(),
        scratch_types=[
            pltpu.VMEM((epw,), jnp.int32),
            pltpu.VMEM((_LSTRIDE * 16,), jnp.float32),
            pltpu.VMEM((_HALF,), jnp.float32),
        ],
        compiler_params=pltpu.CompilerParams(needs_layout_passes=False),
    )
    deg_p = deg_call(dst_flat, zeros_deg).T

    acc_h = _make_acc_call(W1.shape[1], nchunk, ch)
    acc_o = _make_acc_call(W3.shape[1], nchunk, ch)

    h1 = _tc_h1(node_features, W1)
    g1, inv = _tc_first(deg_p, h1)
    a1 = acc_h(g1, src, dst, zeros_h)
    g2 = _tc_mid(a1, g1, inv, b1.reshape(1, -1), W2)
    a2 = acc_h(g2, src, dst, zeros_h)
    g3 = _tc_mid(a2, g2, inv, b2.reshape(1, -1), W3)
    a3 = acc_o(g3, src, dst, zeros_o)
    return _tc_final(a3, g3, inv, b3.reshape(1, -1))
